# baseline (speedup 1.0000x reference)
import functools

import jax
import jax.numpy as jnp
from jax.experimental import pallas as pl
from jax.experimental.pallas import tpu as pltpu


def _body(x_ref, u_ref, wc_ref, lp_ref, o_ref, *, TB, N, U):
    x = x_ref[...]
    u = u_ref[...]
    eq = jax.lax.broadcasted_iota(jnp.int32, (TB, N), 1) == x
    onehot_f = jnp.where(eq, jnp.float32(1.0), jnp.float32(0.0))
    onehot_bf = onehot_f.astype(jnp.bfloat16)
    g = jnp.dot(onehot_bf, wc_ref[...], preferred_element_type=jnp.float32)
    base = jnp.dot(onehot_f, lp_ref[...], preferred_element_type=jnp.float32)
    logits = base
    for d in range(U):
        logits = logits + u[:, d:d + 1] * g[:, d * N:(d + 1) * N]
    m = jnp.max(logits, axis=1, keepdims=True)
    ex = jnp.exp(logits - m)
    lz = jnp.log(jnp.sum(ex, axis=1, keepdims=True)) + m
    o_ref[...] = logits - lz


@functools.partial(jax.jit, static_argnames=("interpret", "tb"))
def kernel(x_curr, u_curr, logP0, W, interpret=False, tb=1024):
    T = x_curr.shape[0]
    N = logP0.shape[0]
    U = u_curr.shape[1]
    cols = jnp.arange(N)[None, :]
    srows = jnp.arange(N)[:, None]
    k = jnp.clip(cols - (cols > srows).astype(jnp.int32), 0, N - 2)
    W64 = jnp.take_along_axis(W, k[:, :, None], axis=1)
    W64 = jnp.where((cols == srows)[:, :, None], 0.0, W64)
    Wcat = W64.transpose(0, 2, 1).reshape(N, U * N).astype(jnp.bfloat16)

    TB = tb
    x2 = x_curr.astype(jnp.int32).reshape(T, 1)
    grid = (T // TB,)
    out = pl.pallas_call(
        functools.partial(_body, TB=TB, N=N, U=U),
        grid=grid,
        in_specs=[
            pl.BlockSpec((TB, 1), lambda i: (i, 0)),
            pl.BlockSpec((TB, U), lambda i: (i, 0)),
            pl.BlockSpec((N, U * N), lambda i: (0, 0)),
            pl.BlockSpec((N, N), lambda i: (0, 0)),
        ],
        out_specs=pl.BlockSpec((TB, N), lambda i: (i, 0)),
        out_shape=jax.ShapeDtypeStruct((T, N), jnp.float32),
        compiler_params=pltpu.CompilerParams(
            dimension_semantics=("arbitrary",),
        ),
        interpret=interpret,
    )(x2, u_curr, Wcat, logP0)
    return out


# speedup vs baseline: 1.9477x; 1.9477x over previous
import functools

import jax
import jax.numpy as jnp
from jax.experimental import pallas as pl
from jax.experimental.pallas import tpu as pltpu


def _body(x_ref, u_ref, wc_ref, lp_ref, o_ref, *, TB, N, U):
    x = x_ref[...]
    u = u_ref[...]
    c = jax.lax.broadcasted_iota(jnp.int32, (TB, N * U), 1)
    mask = (c // U) == x
    u_t = jnp.tile(u, (1, N))
    z = jnp.where(mask, u_t, 0.0).astype(jnp.bfloat16)
    stim = jnp.dot(z, wc_ref[...], preferred_element_type=jnp.float32)
    eq = jax.lax.broadcasted_iota(jnp.int32, (TB, N), 1) == x
    onehot_f = jnp.where(eq, jnp.float32(1.0), jnp.float32(0.0))
    base = jnp.dot(onehot_f, lp_ref[...], preferred_element_type=jnp.float32)
    logits = base + stim
    m = jnp.max(logits, axis=1, keepdims=True)
    ex = jnp.exp(logits - m)
    lz = jnp.log(jnp.sum(ex, axis=1, keepdims=True)) + m
    o_ref[...] = logits - lz


@functools.partial(jax.jit, static_argnames=("interpret", "tb"))
def kernel(x_curr, u_curr, logP0, W, interpret=False, tb=1024):
    T = x_curr.shape[0]
    N = logP0.shape[0]
    U = u_curr.shape[1]
    cols = jnp.arange(N)[None, :]
    srows = jnp.arange(N)[:, None]
    k = jnp.clip(cols - (cols > srows).astype(jnp.int32), 0, N - 2)
    W64 = jnp.take_along_axis(W, k[:, :, None], axis=1)
    W64 = jnp.where((cols == srows)[:, :, None], 0.0, W64)
    Wcat = W64.transpose(0, 2, 1).reshape(N * U, N).astype(jnp.bfloat16)

    TB = tb
    x2 = x_curr.astype(jnp.int32).reshape(T, 1)
    grid = (T // TB,)
    out = pl.pallas_call(
        functools.partial(_body, TB=TB, N=N, U=U),
        grid=grid,
        in_specs=[
            pl.BlockSpec((TB, 1), lambda i: (i, 0)),
            pl.BlockSpec((TB, U), lambda i: (i, 0)),
            pl.BlockSpec((N * U, N), lambda i: (0, 0)),
            pl.BlockSpec((N, N), lambda i: (0, 0)),
        ],
        out_specs=pl.BlockSpec((TB, N), lambda i: (i, 0)),
        out_shape=jax.ShapeDtypeStruct((T, N), jnp.float32),
        compiler_params=pltpu.CompilerParams(
            dimension_semantics=("arbitrary",),
        ),
        interpret=interpret,
    )(x2, u_curr, Wcat, logP0)
    return out


# speedup vs baseline: 2.0551x; 1.0551x over previous
import functools

import jax
import jax.numpy as jnp
from jax.experimental import pallas as pl
from jax.experimental.pallas import tpu as pltpu


def _body(x_ref, u_ref, wc_ref, lp_ref, o_ref, *, TB, N, U):
    x = x_ref[...]
    u = u_ref[...]
    c = jax.lax.broadcasted_iota(jnp.int32, (TB, N * U), 1)
    mask = (c // U) == x
    u_t = jnp.tile(u, (1, N))
    z = jnp.where(mask, u_t, 0.0).astype(jnp.bfloat16)
    stim = jnp.dot(z, wc_ref[...], preferred_element_type=jnp.float32)
    eq = jax.lax.broadcasted_iota(jnp.int32, (TB, N), 1) == x
    onehot_f = jnp.where(eq, jnp.float32(1.0), jnp.float32(0.0))
    base = jnp.dot(onehot_f, lp_ref[...], preferred_element_type=jnp.float32)
    logits = base + stim
    m = jnp.max(logits, axis=1, keepdims=True)
    ex = jnp.exp(logits - m)
    lz = jnp.log(jnp.sum(ex, axis=1, keepdims=True)) + m
    o_ref[...] = logits - lz


@functools.partial(jax.jit, static_argnames=("interpret", "tb"))
def kernel(x_curr, u_curr, logP0, W, interpret=False, tb=2048):
    T = x_curr.shape[0]
    N = logP0.shape[0]
    U = u_curr.shape[1]
    cols = jnp.arange(N)[None, :]
    srows = jnp.arange(N)[:, None]
    k = jnp.clip(cols - (cols > srows).astype(jnp.int32), 0, N - 2)
    W64 = jnp.take_along_axis(W, k[:, :, None], axis=1)
    W64 = jnp.where((cols == srows)[:, :, None], 0.0, W64)
    Wcat = W64.transpose(0, 2, 1).reshape(N * U, N).astype(jnp.bfloat16)

    TB = tb
    x2 = x_curr.astype(jnp.int32).reshape(T, 1)
    grid = (T // TB,)
    out = pl.pallas_call(
        functools.partial(_body, TB=TB, N=N, U=U),
        grid=grid,
        in_specs=[
            pl.BlockSpec((TB, 1), lambda i: (i, 0)),
            pl.BlockSpec((TB, U), lambda i: (i, 0)),
            pl.BlockSpec((N * U, N), lambda i: (0, 0)),
            pl.BlockSpec((N, N), lambda i: (0, 0)),
        ],
        out_specs=pl.BlockSpec((TB, N), lambda i: (i, 0)),
        out_shape=jax.ShapeDtypeStruct((T, N), jnp.float32),
        compiler_params=pltpu.CompilerParams(
            dimension_semantics=("arbitrary",),
        ),
        interpret=interpret,
    )(x2, u_curr, Wcat, logP0)
    return out
